# baseline (speedup 1.0000x reference)
import functools

import jax
import jax.numpy as jnp
from jax.experimental import pallas as pl
from jax.experimental.pallas import tpu as pltpu

E = 8
TOPK = 2
T = 256
EBT = T
NP = E * EBT
P2 = TOPK * T


def _routing(logits):
    m = jnp.max(logits, axis=1, keepdims=True)
    p = jnp.exp(logits - m)
    p = p / jnp.sum(p, axis=1, keepdims=True)

    eidx = jax.lax.broadcasted_iota(jnp.int32, (T, E), 1)
    m1 = jnp.max(p, axis=1, keepdims=True)
    a1 = jnp.min(jnp.where(p == m1, eidx, E), axis=1, keepdims=True)
    p2 = jnp.where(eidx == a1, -1.0, p)
    m2 = jnp.max(p2, axis=1, keepdims=True)
    a2 = jnp.min(jnp.where(p2 == m2, eidx, E), axis=1, keepdims=True)
    s = m1 + m2
    w1 = m1 / s
    w2 = m2 / s

    e_pairs = jnp.concatenate([a1, a2], axis=0)
    w_pairs = jnp.concatenate([w1, w2], axis=0)
    tio = jax.lax.broadcasted_iota(jnp.int32, (T, 1), 0).astype(jnp.float32)
    t_pairs = jnp.concatenate([tio, tio], axis=0)

    oh = (e_pairs == jax.lax.broadcasted_iota(jnp.int32, (P2, E), 1))
    ohf = oh.astype(jnp.float32)

    pr = jax.lax.broadcasted_iota(jnp.int32, (P2, P2), 0)
    pc = jax.lax.broadcasted_iota(jnp.int32, (P2, P2), 1)
    lt = (pc <= pr).astype(jnp.float32)
    incl = jnp.dot(lt, ohf, preferred_element_type=jnp.float32)
    rank = jnp.sum((incl - 1.0) * ohf, axis=1, keepdims=True)
    pos = e_pairs.astype(jnp.float32) * EBT + rank

    slot_iota = jax.lax.broadcasted_iota(jnp.int32, (P2, NP), 1).astype(
        jnp.float32)
    at = (pos == slot_iota).astype(jnp.float32)
    tw = jnp.concatenate([t_pairs, w_pairs], axis=1)
    cdims = (((0,), (0,)), ((), ()))
    return jax.lax.dot_general(
        at, tw, cdims, preferred_element_type=jnp.float32)


def _moe_kernel(g_ref, x_ref, ga_ref, gb_ref, da_ref, db_ref, out_ref,
                idw_ref, *, ff):
    e = pl.program_id(0)

    @pl.when(e == 0)
    def _():
        idw_ref[...] = _routing(g_ref[...])

    idw = idw_ref[pl.ds(e * EBT, EBT), :]
    ids = idw[:, 0:1]
    w = idw[:, 1:2]
    tcol = jax.lax.broadcasted_iota(jnp.int32, (EBT, T), 1).astype(
        jnp.float32)
    perm = (ids == tcol).astype(jnp.float32)
    xg = jnp.dot(perm, x_ref[...],
                 preferred_element_type=jnp.float32)
    xb = xg.astype(jnp.bfloat16)
    cdims = (((1,), (1,)), ((), ()))
    g = jax.lax.dot_general(
        xb, ga_ref[0], cdims,
        preferred_element_type=jnp.float32)
    u = jax.lax.dot_general(
        xb, gb_ref[0], cdims,
        preferred_element_type=jnp.float32)
    h = (g * jax.lax.logistic(g) * u).astype(jnp.bfloat16)
    ya = jax.lax.dot_general(
        h, da_ref[0], cdims,
        preferred_element_type=jnp.float32)
    yb = jax.lax.dot_general(
        h, db_ref[0], cdims,
        preferred_element_type=jnp.float32)
    y = jnp.concatenate([ya, yb], axis=1) * w
    sdims = (((0,), (0,)), ((), ()))
    contrib = jax.lax.dot_general(
        perm, y, sdims, preferred_element_type=jnp.float32)

    @pl.when(e == 0)
    def _():
        out_ref[...] = contrib

    @pl.when(e > 0)
    def _():
        out_ref[...] += contrib


def kernel(x, gating_output, gate_up_proj, down_proj):
    t, d = x.shape
    ff2 = gate_up_proj.shape[1]
    ff = ff2 // 2
    dh = d // 2

    out = pl.pallas_call(
        functools.partial(_moe_kernel, ff=ff),
        grid=(E,),
        in_specs=[
            pl.BlockSpec((t, E), lambda e: (0, 0)),
            pl.BlockSpec((t, d), lambda e: (0, 0)),
            pl.BlockSpec((1, ff, d), lambda e: (e, 0, 0)),
            pl.BlockSpec((1, ff, d), lambda e: (e, 1, 0)),
            pl.BlockSpec((1, dh, ff), lambda e: (e, 0, 0)),
            pl.BlockSpec((1, dh, ff), lambda e: (e, 1, 0)),
        ],
        out_specs=pl.BlockSpec((t, d), lambda e: (0, 0)),
        scratch_shapes=[
            pltpu.VMEM((NP, 2), jnp.float32),
        ],
        out_shape=jax.ShapeDtypeStruct((t, d), jnp.float32),
    )(gating_output, x, gate_up_proj, gate_up_proj, down_proj, down_proj)
    return out


# speedup vs baseline: 1.1478x; 1.1478x over previous
import functools

import jax
import jax.numpy as jnp
from jax.experimental import pallas as pl
from jax.experimental.pallas import tpu as pltpu

E = 8
TOPK = 2
T = 256
EBT = T
NP = E * EBT
P2 = TOPK * T


def _routing(logits):
    m = jnp.max(logits, axis=1, keepdims=True)
    p = jnp.exp(logits - m)
    p = p / jnp.sum(p, axis=1, keepdims=True)

    eidx = jax.lax.broadcasted_iota(jnp.int32, (T, E), 1)
    m1 = jnp.max(p, axis=1, keepdims=True)
    a1 = jnp.min(jnp.where(p == m1, eidx, E), axis=1, keepdims=True)
    p2 = jnp.where(eidx == a1, -1.0, p)
    m2 = jnp.max(p2, axis=1, keepdims=True)
    a2 = jnp.min(jnp.where(p2 == m2, eidx, E), axis=1, keepdims=True)
    s = m1 + m2
    w1 = m1 / s
    w2 = m2 / s

    e_pairs = jnp.concatenate([a1, a2], axis=0)
    w_pairs = jnp.concatenate([w1, w2], axis=0)
    tio = jax.lax.broadcasted_iota(jnp.int32, (T, 1), 0).astype(jnp.float32)
    t_pairs = jnp.concatenate([tio, tio], axis=0)

    oh = (e_pairs == jax.lax.broadcasted_iota(jnp.int32, (P2, E), 1))
    ohf = oh.astype(jnp.float32)

    pr = jax.lax.broadcasted_iota(jnp.int32, (P2, P2), 0)
    pc = jax.lax.broadcasted_iota(jnp.int32, (P2, P2), 1)
    lt = (pc <= pr).astype(jnp.float32)
    incl = jnp.dot(lt, ohf, preferred_element_type=jnp.float32)
    rank = jnp.sum((incl - 1.0) * ohf, axis=1, keepdims=True)
    pos = e_pairs.astype(jnp.float32) * EBT + rank

    slot_iota = jax.lax.broadcasted_iota(jnp.int32, (P2, NP), 1).astype(
        jnp.float32)
    at = (pos == slot_iota).astype(jnp.float32)
    tw = jnp.concatenate([t_pairs, w_pairs], axis=1)
    cdims = (((0,), (0,)), ((), ()))
    return jax.lax.dot_general(
        at, tw, cdims, preferred_element_type=jnp.float32)


def _moe_kernel(g_ref, x_ref, gu_hbm, dn_ref, out_ref,
                idw_ref, gu_buf, gu_sem, *, ff):
    e = pl.program_id(0)

    def gu_copy(ee, half):
        sl = pl.ds(half * ff, ff)
        return pltpu.make_async_copy(gu_hbm.at[ee, sl],
                                     gu_buf.at[ee % 2, sl],
                                     gu_sem.at[ee % 2, half])

    @pl.when(e == 0)
    def _():
        gu_copy(0, 0).start()
        gu_copy(0, 1).start()
        gu_copy(1, 0).start()
        gu_copy(1, 1).start()
        idw_ref[...] = _routing(g_ref[...])

    slot = e % 2
    idw = idw_ref[pl.ds(e * EBT, EBT), :]
    ids = idw[:, 0:1]
    w = idw[:, 1:2]
    tcol = jax.lax.broadcasted_iota(jnp.int32, (EBT, T), 1).astype(
        jnp.float32)
    perm = (ids == tcol).astype(jnp.float32)
    xg = jnp.dot(perm, x_ref[...],
                 preferred_element_type=jnp.float32)
    xb = xg.astype(jnp.bfloat16)
    cdims = (((1,), (1,)), ((), ()))
    gu_copy(e, 0).wait()
    g = jax.lax.dot_general(
        xb, gu_buf[slot, :ff], cdims,
        preferred_element_type=jnp.float32)
    gu_copy(e, 1).wait()
    u = jax.lax.dot_general(
        xb, gu_buf[slot, ff:], cdims,
        preferred_element_type=jnp.float32)

    @pl.when(e < E - 2)
    def _():
        gu_copy(e + 2, 0).start()
        gu_copy(e + 2, 1).start()

    h = (g * jax.lax.logistic(g) * u).astype(jnp.bfloat16)
    y = jax.lax.dot_general(
        h, dn_ref[0], cdims,
        preferred_element_type=jnp.float32)
    y = y * w
    sdims = (((0,), (0,)), ((), ()))
    contrib = jax.lax.dot_general(
        perm, y, sdims, preferred_element_type=jnp.float32)

    @pl.when(e == 0)
    def _():
        out_ref[...] = contrib

    @pl.when(e > 0)
    def _():
        out_ref[...] += contrib


def kernel(x, gating_output, gate_up_proj, down_proj):
    t, d = x.shape
    ff2 = gate_up_proj.shape[1]
    ff = ff2 // 2

    out = pl.pallas_call(
        functools.partial(_moe_kernel, ff=ff),
        grid=(E,),
        in_specs=[
            pl.BlockSpec((t, E), lambda e: (0, 0)),
            pl.BlockSpec((t, d), lambda e: (0, 0)),
            pl.BlockSpec(memory_space=pltpu.MemorySpace.HBM),
            pl.BlockSpec((1, d, ff), lambda e: (e, 0, 0)),
        ],
        out_specs=pl.BlockSpec((t, d), lambda e: (0, 0)),
        scratch_shapes=[
            pltpu.VMEM((NP, 2), jnp.float32),
            pltpu.VMEM((2, ff2, d), jnp.float32),
            pltpu.SemaphoreType.DMA((2, 2)),
        ],
        out_shape=jax.ShapeDtypeStruct((t, d), jnp.float32),
    )(gating_output, x, gate_up_proj, down_proj)
    return out


# speedup vs baseline: 1.1500x; 1.0019x over previous
import functools

import jax
import jax.numpy as jnp
from jax.experimental import pallas as pl
from jax.experimental.pallas import tpu as pltpu

E = 8
TOPK = 2
T = 256
EBT = T
NP = E * EBT
P2 = TOPK * T


def _routing(logits):
    m = jnp.max(logits, axis=1, keepdims=True)
    p = jnp.exp(logits - m)
    p = p / jnp.sum(p, axis=1, keepdims=True)

    eidx = jax.lax.broadcasted_iota(jnp.int32, (T, E), 1)
    m1 = jnp.max(p, axis=1, keepdims=True)
    a1 = jnp.min(jnp.where(p == m1, eidx, E), axis=1, keepdims=True)
    p2 = jnp.where(eidx == a1, -1.0, p)
    m2 = jnp.max(p2, axis=1, keepdims=True)
    a2 = jnp.min(jnp.where(p2 == m2, eidx, E), axis=1, keepdims=True)
    s = m1 + m2
    w1 = m1 / s
    w2 = m2 / s

    e_pairs = jnp.concatenate([a1, a2], axis=0)
    w_pairs = jnp.concatenate([w1, w2], axis=0)
    tio = jax.lax.broadcasted_iota(jnp.int32, (T, 1), 0).astype(jnp.float32)
    t_pairs = jnp.concatenate([tio, tio], axis=0)

    oh = (e_pairs == jax.lax.broadcasted_iota(jnp.int32, (P2, E), 1))
    ohf = oh.astype(jnp.float32)

    pr = jax.lax.broadcasted_iota(jnp.int32, (P2, P2), 0)
    pc = jax.lax.broadcasted_iota(jnp.int32, (P2, P2), 1)
    lt = (pc <= pr).astype(jnp.float32)
    incl = jnp.dot(lt, ohf, preferred_element_type=jnp.float32)
    rank = jnp.sum((incl - 1.0) * ohf, axis=1, keepdims=True)
    pos = e_pairs.astype(jnp.float32) * EBT + rank

    slot_iota = jax.lax.broadcasted_iota(jnp.int32, (P2, NP), 1).astype(
        jnp.float32)
    at = (pos == slot_iota).astype(jnp.float32)
    tw = jnp.concatenate([t_pairs, w_pairs], axis=1)
    cdims = (((0,), (0,)), ((), ()))
    return jax.lax.dot_general(
        at, tw, cdims, preferred_element_type=jnp.float32)


def _moe_kernel(g_ref, x_ref, gu_hbm, dn_ref, out_ref,
                idw_ref, gu_buf, gu_sem, *, ff):
    e = pl.program_id(0)

    def gu_copy(ee, half):
        sl = pl.ds(half * ff, ff)
        return pltpu.make_async_copy(gu_hbm.at[ee, sl],
                                     gu_buf.at[ee % 2, sl],
                                     gu_sem.at[ee % 2, half])

    @pl.when(e == 0)
    def _():
        gu_copy(0, 0).start()
        gu_copy(0, 1).start()
        gu_copy(1, 0).start()
        gu_copy(1, 1).start()
        idw_ref[...] = _routing(g_ref[...])

    slot = e % 2
    idw = idw_ref[pl.ds(e * EBT, EBT), :]
    ids = idw[:, 0:1]
    w = idw[:, 1:2]
    tcol = jax.lax.broadcasted_iota(jnp.int32, (EBT, T), 1).astype(
        jnp.float32)
    permf = (ids == tcol).astype(jnp.float32)
    perm = permf.astype(jnp.bfloat16)
    xb = jnp.dot(perm, x_ref[...].astype(jnp.bfloat16),
                 preferred_element_type=jnp.float32
                 ).astype(jnp.bfloat16)
    cdims = (((1,), (1,)), ((), ()))
    gu_copy(e, 0).wait()
    g = jax.lax.dot_general(
        xb, gu_buf[slot, :ff], cdims,
        preferred_element_type=jnp.float32)
    gu_copy(e, 1).wait()
    u = jax.lax.dot_general(
        xb, gu_buf[slot, ff:], cdims,
        preferred_element_type=jnp.float32)

    @pl.when(e < E - 2)
    def _():
        gu_copy(e + 2, 0).start()
        gu_copy(e + 2, 1).start()

    h = (g * jax.lax.logistic(g) * u).astype(jnp.bfloat16)
    y = jax.lax.dot_general(
        h, dn_ref[0], cdims,
        preferred_element_type=jnp.float32)
    y = y * w
    sdims = (((0,), (0,)), ((), ()))
    contrib = jax.lax.dot_general(
        permf, y, sdims, preferred_element_type=jnp.float32)

    @pl.when(e == 0)
    def _():
        out_ref[...] = contrib

    @pl.when(e > 0)
    def _():
        out_ref[...] += contrib


def kernel(x, gating_output, gate_up_proj, down_proj):
    t, d = x.shape
    ff2 = gate_up_proj.shape[1]
    ff = ff2 // 2

    out = pl.pallas_call(
        functools.partial(_moe_kernel, ff=ff),
        grid=(E,),
        in_specs=[
            pl.BlockSpec((t, E), lambda e: (0, 0)),
            pl.BlockSpec((t, d), lambda e: (0, 0)),
            pl.BlockSpec(memory_space=pltpu.MemorySpace.HBM),
            pl.BlockSpec((1, d, ff), lambda e: (e, 0, 0)),
        ],
        out_specs=pl.BlockSpec((t, d), lambda e: (0, 0)),
        scratch_shapes=[
            pltpu.VMEM((NP, 2), jnp.float32),
            pltpu.VMEM((2, ff2, d), jnp.float32),
            pltpu.SemaphoreType.DMA((2, 2)),
        ],
        out_shape=jax.ShapeDtypeStruct((t, d), jnp.float32),
    )(gating_output, x, gate_up_proj, down_proj)
    return out
